# baseline (speedup 1.0000x reference)
import jax
import jax.numpy as jnp
from jax.experimental import pallas as pl
from jax.experimental.pallas import tpu as pltpu

B = 4096
V = 6890
BB = 512


def _body(idx_ref, vt_ref, jt_ref, j26_ref, out_ref):
    plane = vt_ref[0]
    out_ref[0, :24, :] = jt_ref[0]
    for j in range(11):
        out_ref[0, 24 + j, :] = vt_ref[0, idx_ref[j], :]
    out_ref[0, 35:61, :] = jnp.dot(j26_ref[...], plane,
                                   preferred_element_type=jnp.float32)


def kernel(vertices, joints, extra_joints_idxs, J_regressor_extra9,
           J_regressor_h36m17):
    vt = jnp.transpose(vertices, (2, 1, 0))
    jt = jnp.transpose(joints, (2, 1, 0))
    j26 = jnp.concatenate([J_regressor_extra9, J_regressor_h36m17], axis=0)

    grid_spec = pltpu.PrefetchScalarGridSpec(
        num_scalar_prefetch=1,
        grid=(3, B // BB),
        in_specs=[
            pl.BlockSpec((1, V, BB), lambda k, b, *_: (k, 0, b)),
            pl.BlockSpec((1, 24, BB), lambda k, b, *_: (k, 0, b)),
            pl.BlockSpec((26, V), lambda k, b, *_: (0, 0)),
        ],
        out_specs=pl.BlockSpec((1, 61, BB), lambda k, b, *_: (k, 0, b)),
    )

    out_t = pl.pallas_call(
        _body,
        grid_spec=grid_spec,
        out_shape=jax.ShapeDtypeStruct((3, 61, B), jnp.float32),
    )(extra_joints_idxs, vt, jt, j26)

    return jnp.transpose(out_t, (2, 1, 0))


# speedup vs baseline: 1.0068x; 1.0068x over previous
import jax
import jax.numpy as jnp
from jax.experimental import pallas as pl
from jax.experimental.pallas import tpu as pltpu

B = 4096
V = 6890
BB = 256


def _body(idx_ref, vt_ref, jt_ref, j26_ref, out_ref):
    plane = vt_ref[0]
    out_ref[0, :24, :] = jt_ref[0]
    for j in range(11):
        out_ref[0, 24 + j, :] = vt_ref[0, idx_ref[j], :]
    out_ref[0, 35:61, :] = jnp.dot(j26_ref[...], plane,
                                   preferred_element_type=jnp.float32)


def kernel(vertices, joints, extra_joints_idxs, J_regressor_extra9,
           J_regressor_h36m17):
    vt = jnp.transpose(vertices, (2, 1, 0))
    jt = jnp.transpose(joints, (2, 1, 0))
    j26 = jnp.concatenate([J_regressor_extra9, J_regressor_h36m17], axis=0)

    grid_spec = pltpu.PrefetchScalarGridSpec(
        num_scalar_prefetch=1,
        grid=(3, B // BB),
        in_specs=[
            pl.BlockSpec((1, V, BB), lambda k, b, *_: (k, 0, b)),
            pl.BlockSpec((1, 24, BB), lambda k, b, *_: (k, 0, b)),
            pl.BlockSpec((26, V), lambda k, b, *_: (0, 0)),
        ],
        out_specs=pl.BlockSpec((1, 61, BB), lambda k, b, *_: (k, 0, b)),
    )

    out_t = pl.pallas_call(
        _body,
        grid_spec=grid_spec,
        out_shape=jax.ShapeDtypeStruct((3, 61, B), jnp.float32),
    )(extra_joints_idxs, vt, jt, j26)

    return jnp.transpose(out_t, (2, 1, 0))


# speedup vs baseline: 1.0572x; 1.0501x over previous
import jax
import jax.numpy as jnp
from jax.experimental import pallas as pl
from jax.experimental.pallas import tpu as pltpu

B = 4096
V = 6890
BB = 256


def _body(idx_ref, vt_ref, jt_ref, j26_ref, out_ref):
    plane = vt_ref[0]
    out_ref[0, :24, :] = jt_ref[0]
    for j in range(11):
        out_ref[0, 24 + j, :] = vt_ref[0, idx_ref[j], :]
    out_ref[0, 35:61, :] = jnp.dot(j26_ref[...], plane,
                                   preferred_element_type=jnp.float32)


def kernel(vertices, joints, extra_joints_idxs, J_regressor_extra9,
           J_regressor_h36m17):
    vt = jnp.transpose(vertices, (2, 1, 0))
    jt = jnp.transpose(joints, (2, 1, 0))
    j26 = jnp.concatenate([J_regressor_extra9, J_regressor_h36m17], axis=0)

    grid_spec = pltpu.PrefetchScalarGridSpec(
        num_scalar_prefetch=1,
        grid=(3, B // BB),
        in_specs=[
            pl.BlockSpec((1, V, BB), lambda k, b, *_: (k, 0, b)),
            pl.BlockSpec((1, 24, BB), lambda k, b, *_: (k, 0, b)),
            pl.BlockSpec((26, V), lambda k, b, *_: (0, 0)),
        ],
        out_specs=pl.BlockSpec((1, 61, BB), lambda k, b, *_: (k, 0, b)),
    )

    out_t = pl.pallas_call(
        _body,
        grid_spec=grid_spec,
        out_shape=jax.ShapeDtypeStruct((3, 61, B), jnp.float32),
        compiler_params=pltpu.CompilerParams(
            dimension_semantics=("parallel", "parallel")),
    )(extra_joints_idxs, vt, jt, j26)

    return jnp.transpose(out_t, (2, 1, 0))
